# baseline (speedup 1.0000x reference)
import functools

import jax
import jax.numpy as jnp
from jax import lax
from jax.experimental import pallas as pl
from jax.experimental.pallas import tpu as pltpu
from jax.experimental.pallas import tpu_sc as plsc

VOCAB = 100000
D_MODEL = 128
BATCH = 4096
HIST = 50

NC = 2
NS = 16
NW = NC * NS
B_PER_W = BATCH // NW
NCH = B_PER_W
NBUF = 8


def _emb_body(x_hbm, table_hbm, out_hbm, idx_v, *scratch):
    bufs = scratch[:NBUF]
    sems_g = scratch[NBUF:2 * NBUF]
    sems_s = scratch[2 * NBUF:3 * NBUF]
    wid = lax.axis_index("s") * NC + lax.axis_index("c")
    b0 = wid * B_PER_W
    pltpu.sync_copy(x_hbm.at[pl.ds(b0, B_PER_W)], idx_v)

    for b in range(NBUF):
        pltpu.async_copy(table_hbm.at[idx_v.at[b]], bufs[b], sems_g[b])

    def outer(i, carry):
        base = i * NBUF
        for b in range(NBUF):
            j = base + b
            pltpu.make_async_copy(table_hbm.at[idx_v.at[j]], bufs[b], sems_g[b]).wait()
            pltpu.async_copy(bufs[b], out_hbm.at[b0 + j], sems_s[b])

            @pl.when(j + NBUF < NCH)
            def _(j=j, b=b):
                pltpu.make_async_copy(bufs[b], out_hbm.at[b0 + j], sems_s[b]).wait()
                pltpu.async_copy(table_hbm.at[idx_v.at[j + NBUF]], bufs[b], sems_g[b])

        return carry

    lax.fori_loop(0, NCH // NBUF, outer, 0)

    for b in range(NBUF):
        j = NCH - NBUF + b
        pltpu.make_async_copy(bufs[b], out_hbm.at[b0 + j], sems_s[b]).wait()


_emb = functools.partial(
    pl.kernel,
    mesh=plsc.VectorSubcoreMesh(core_axis_name="c", subcore_axis_name="s"),
    out_type=jax.ShapeDtypeStruct((BATCH, HIST, D_MODEL), jnp.float32),
    compiler_params=pltpu.CompilerParams(use_tc_tiling_on_sc=True),
    scratch_types=(
        [pltpu.VMEM((B_PER_W, HIST), jnp.int32)]
        + [pltpu.VMEM((HIST, D_MODEL), jnp.float32) for _ in range(NBUF)]
        + [pltpu.SemaphoreType.DMA for _ in range(2 * NBUF)]
    ),
)(_emb_body)


def kernel(x, table):
    return _emb(x.astype(jnp.int32), table)


# speedup vs baseline: 1.7977x; 1.7977x over previous
import functools

import jax
import jax.numpy as jnp
from jax import lax
from jax.experimental import pallas as pl
from jax.experimental.pallas import tpu as pltpu
from jax.experimental.pallas import tpu_sc as plsc

VOCAB = 100000
D_MODEL = 128
BATCH = 4096
HIST = 50

NC = 2
NS = 16
NW = NC * NS
B_PER_W = BATCH // NW
NCH = HIST
NBUF = 5


def _emb_body(xt_hbm, table_hbm, out_hbm, idx_v, *scratch):
    bufs = scratch[:NBUF]
    sems_g = scratch[NBUF:2 * NBUF]
    sems_s = scratch[2 * NBUF:3 * NBUF]
    wid = lax.axis_index("s") * NC + lax.axis_index("c")
    b0 = wid * B_PER_W
    pltpu.sync_copy(xt_hbm.at[:, pl.ds(b0, B_PER_W)], idx_v)

    for b in range(NBUF):
        pltpu.async_copy(table_hbm.at[idx_v.at[b]], bufs[b], sems_g[b])

    def outer(i, carry):
        base = i * NBUF
        for b in range(NBUF):
            j = base + b
            pltpu.make_async_copy(table_hbm.at[idx_v.at[j]], bufs[b], sems_g[b]).wait()
            pltpu.async_copy(bufs[b], out_hbm.at[j, pl.ds(b0, B_PER_W)], sems_s[b])

            @pl.when(j + NBUF < NCH)
            def _(j=j, b=b):
                pltpu.make_async_copy(
                    bufs[b], out_hbm.at[j, pl.ds(b0, B_PER_W)], sems_s[b]
                ).wait()
                pltpu.async_copy(table_hbm.at[idx_v.at[j + NBUF]], bufs[b], sems_g[b])

        return carry

    lax.fori_loop(0, NCH // NBUF, outer, 0)

    for b in range(NBUF):
        j = NCH - NBUF + b
        pltpu.make_async_copy(
            bufs[b], out_hbm.at[j, pl.ds(b0, B_PER_W)], sems_s[b]
        ).wait()


_emb = functools.partial(
    pl.kernel,
    mesh=plsc.VectorSubcoreMesh(core_axis_name="c", subcore_axis_name="s"),
    out_type=jax.ShapeDtypeStruct((HIST, BATCH, D_MODEL), jnp.float32),
    scratch_types=(
        [pltpu.VMEM((HIST, B_PER_W), jnp.int32)]
        + [pltpu.VMEM((B_PER_W, D_MODEL), jnp.float32) for _ in range(NBUF)]
        + [pltpu.SemaphoreType.DMA for _ in range(2 * NBUF)]
    ),
)(_emb_body)


def kernel(x, table):
    xt = x.astype(jnp.int32).T
    out = _emb(xt, table)
    return jnp.transpose(out, (1, 0, 2))
